# baseline (speedup 1.0000x reference)
import functools

import jax
import jax.numpy as jnp
import numpy as np
from jax import lax
from jax.experimental import pallas as pl
from jax.experimental.pallas import tpu as pltpu
from jax.experimental.pallas import tpu_sc as plsc

_R = 128
_N = 32768
_K = 64
_L = 16
_NV = _N // _L
_NC = 2
_NS = 16
_NW = _NC * _NS
_RPW = _R // _NW
_G = 8
_NG = _NV // _G
_MV = _NG * _L
_CAP = 512
_CV = _CAP // _L
_COMPACT_AT = _CAP - _G * _L

_NEG_INF = np.float32("-inf")


def _mono_u32(v):
    i = plsc.bitcast(v, jnp.int32)
    flip = lax.shift_right_arithmetic(i, 31) & jnp.int32(0x7FFFFFFF)
    mi = i ^ flip
    return plsc.bitcast(mi, jnp.uint32) ^ jnp.uint32(0x80000000)


def _unmono_f32(mu_vec):
    mi = plsc.bitcast(mu_vec ^ jnp.uint32(0x80000000), jnp.int32)
    flip = lax.shift_right_arithmetic(mi, 31) & jnp.int32(0x7FFFFFFF)
    return plsc.bitcast(mi ^ flip, jnp.float32)


def _body(x_hbm, out_hbm, row_v, max_v, cand_v, mono_v):
    cid = lax.axis_index("c")
    sid = lax.axis_index("s")
    wid = sid * _NC + cid
    lane = jnp.arange(_L, dtype=jnp.int32)

    def count_ge(cand, off, thr_m, m):
        cand_s = jnp.full((_L,), cand, jnp.uint32)
        nv = (off + _L - 1) // _L

        def cbody(j, acc):
            vals = mono_v[pl.ds(j * _L, _L)]
            valid = (lane + j * _L) < off
            ge = jnp.logical_and(vals >= cand_s, valid)
            return acc + jnp.where(ge, jnp.int32(1), jnp.int32(0))

        accv = lax.fori_loop(0, nv, cbody, jnp.zeros((_L,), jnp.int32))
        cnt = jnp.sum(accv)
        return cnt + jnp.where(cand <= thr_m, m, jnp.int32(0))

    def monoize(off):
        nv = (off + _L - 1) // _L

        def mbody(j, _):
            mono_v[pl.ds(j * _L, _L)] = _mono_u32(cand_v[pl.ds(j * _L, _L)])
            return _

        lax.fori_loop(0, nv, mbody, jnp.int32(0))

    def bisect(off, thr_m, m):
        def bit_body(i, t):
            bit = jnp.uint32(31) - i.astype(jnp.uint32)
            cand = t | lax.shift_left(jnp.uint32(1), bit)
            cnt = count_ge(cand, off, thr_m, m)
            return jnp.where(cnt >= _K, cand, t)

        return lax.fori_loop(0, 32, bit_body, jnp.uint32(0))

    def compact(off, thr_f, thr_m, m):
        monoize(off)
        t = bisect(off, thr_m, m)
        t_s = jnp.full((_L,), t, jnp.uint32)
        nv = (off + _L - 1) // _L

        def rbody(j, noff):
            mono_vals = mono_v[pl.ds(j * _L, _L)]
            vals = cand_v[pl.ds(j * _L, _L)]
            keep = jnp.logical_and(mono_vals > t_s, (lane + j * _L) < off)
            cnt = plsc.all_reduce_population_count(keep)[0]
            plsc.store_compressed(cand_v.at[pl.ds(noff, _L)], vals, mask=keep)
            return noff + cnt

        new_off = lax.fori_loop(0, nv, rbody, jnp.int32(0))
        new_thr_f = _unmono_f32(jnp.full((_L,), t, jnp.uint32))[0]
        return new_off, new_thr_f, t, jnp.int32(_K) - new_off

    def filter_group(src, base, off, thr_vec):
        for u in range(_G):
            v = src[pl.ds((base + u) * _L, _L)]
            pmask = v > thr_vec
            cnt = plsc.all_reduce_population_count(pmask)[0]
            plsc.store_compressed(cand_v.at[pl.ds(off, _L)], v, mask=pmask)
            off = off + cnt
        return off

    def maybe_compact(off, thr_f, thr_m, m):
        return lax.cond(off >= _COMPACT_AT, compact,
                        lambda o, tf, tm, mm: (o, tf, tm, mm),
                        off, thr_f, thr_m, m)

    for rr in range(_RPW):
        row = wid * _RPW + rr
        pltpu.sync_copy(x_hbm.at[row], row_v)

        @plsc.parallel_loop(0, _NG, step=1, unroll=2)
        def max_loop(g):
            acc = row_v[pl.ds(g * _G * _L, _L)]
            for u in range(1, _G):
                acc = jnp.maximum(acc, row_v[pl.ds((g * _G + u) * _L, _L)])
            max_v[pl.ds(g * _L, _L)] = acc

        def b_group(g, carry):
            off, thr_f, thr_m, m = carry
            thr_vec = jnp.full((_L,), thr_f, jnp.float32)
            off = filter_group(max_v, g * _G, off, thr_vec)
            return maybe_compact(off, thr_f, thr_m, m)

        off, thr_f, thr_m, m = lax.fori_loop(
            0, _MV // _L // _G, b_group,
            (jnp.int32(0), jnp.float32(_NEG_INF), jnp.uint32(0),
             jnp.int32(0)))
        monoize(off)
        s_mono = bisect(off, thr_m, m)
        s_f = _unmono_f32(jnp.full((_L,), s_mono, jnp.uint32))[0]

        def c_group(g, carry):
            off, thr_f, thr_m, m = carry
            thr_vec = jnp.full((_L,), thr_f, jnp.float32)
            mvec = max_v[pl.ds(g * _L, _L)]
            hit = plsc.all_reduce_population_count(mvec > thr_vec)[0] > 0

            def slow(o, tf, tm, mm):
                o = filter_group(row_v, g * _G, o, thr_vec)
                return maybe_compact(o, tf, tm, mm)

            return lax.cond(hit, slow, lambda o, tf, tm, mm: (o, tf, tm, mm),
                            off, thr_f, thr_m, m)

        off, thr_f, thr_m, m = lax.fori_loop(
            0, _NG, c_group, (jnp.int32(0), s_f, s_mono, jnp.int32(_K)))

        monoize(off)
        t_row = bisect(off, thr_m, m)
        tf = _unmono_f32(jnp.full((_L,), t_row, jnp.uint32))

        @plsc.parallel_loop(0, _NV, step=1, unroll=8)
        def mask_loop(j):
            v = row_v[pl.ds(j * _L, _L)]
            row_v[pl.ds(j * _L, _L)] = jnp.where(v >= tf, v, _NEG_INF)

        pltpu.sync_copy(row_v, out_hbm.at[row])


@jax.jit
def kernel(x):
    mesh = plsc.VectorSubcoreMesh(
        core_axis_name="c", subcore_axis_name="s",
        num_cores=_NC, num_subcores=_NS)
    run = pl.kernel(
        _body,
        out_type=jax.ShapeDtypeStruct((_R, _N), jnp.float32),
        mesh=mesh,
        scratch_types=[
            pltpu.VMEM((_N,), jnp.float32),
            pltpu.VMEM((_MV,), jnp.float32),
            pltpu.VMEM((_CAP,), jnp.float32),
            pltpu.VMEM((_CAP,), jnp.uint32),
        ],
        compiler_params=pltpu.CompilerParams(needs_layout_passes=False),
    )
    return run(x)


# speedup vs baseline: 1.1006x; 1.1006x over previous
import functools

import jax
import jax.numpy as jnp
import numpy as np
from jax import lax
from jax.experimental import pallas as pl
from jax.experimental.pallas import tpu as pltpu
from jax.experimental.pallas import tpu_sc as plsc

_R = 128
_N = 32768
_K = 64
_L = 16
_NV = _N // _L
_NC = 2
_NS = 16
_NW = _NC * _NS
_RPW = _R // _NW
_G = 8
_NG = _NV // _G
_MV = _NG * _L
_CAP = 512
_COMPACT_AT = _CAP - _G * _L

_NEG_INF = np.float32("-inf")


def _mono_u32(v):
    i = plsc.bitcast(v, jnp.int32)
    flip = lax.shift_right_arithmetic(i, 31) & jnp.int32(0x7FFFFFFF)
    mi = i ^ flip
    return plsc.bitcast(mi, jnp.uint32) ^ jnp.uint32(0x80000000)


def _unmono_f32(mu_vec):
    mi = plsc.bitcast(mu_vec ^ jnp.uint32(0x80000000), jnp.int32)
    flip = lax.shift_right_arithmetic(mi, 31) & jnp.int32(0x7FFFFFFF)
    return plsc.bitcast(mi ^ flip, jnp.float32)


def _body(x_hbm, out_hbm, row_v, max_v, cand_v, mono_v):
    cid = lax.axis_index("c")
    sid = lax.axis_index("s")
    wid = sid * _NC + cid

    neginf_v = jnp.full((_L,), _NEG_INF, jnp.float32)
    zero_u_v = jnp.zeros((_L,), jnp.uint32)
    zero_i_v = jnp.zeros((_L,), jnp.int32)
    k_v = jnp.full((_L,), _K, jnp.int32)

    def monoize(off):
        nv = (off + _L - 1) // _L

        def mbody(j, _):
            mono_v[pl.ds(j * _L, _L)] = _mono_u32(cand_v[pl.ds(j * _L, _L)])
            return _

        lax.fori_loop(0, nv, mbody, jnp.int32(0))
        mono_v[pl.ds(off, _L)] = zero_u_v

    def bisect_v(off, thrm_v, m_v):
        monoize(off)
        nv = (off + _L - 1) // _L
        one_v = jnp.full((_L,), jnp.uint32(1))

        def bit_body(i, t_v):
            shift = jnp.full((_L,), 31 - i, jnp.int32).astype(jnp.uint32)
            cand_vv = t_v | lax.shift_left(one_v, shift)

            def cbody(j, acc):
                ge = mono_v[pl.ds(j * _L, _L)] >= cand_vv
                return acc + plsc.all_reduce_population_count(ge)

            cnt = lax.fori_loop(0, nv, cbody, zero_i_v)
            cnt = cnt + jnp.where(cand_vv <= thrm_v, m_v, zero_i_v)
            return jnp.where(cnt >= k_v, cand_vv, t_v)

        return lax.fori_loop(0, 32, bit_body, zero_u_v)

    def exact_compact(off, thrf_v, thrm_v, m_v):
        t_v = bisect_v(off, thrm_v, m_v)

        def rbody(j, noff):
            keep = mono_v[pl.ds(j * _L, _L)] > t_v
            vals = cand_v[pl.ds(j * _L, _L)]
            cnt = plsc.all_reduce_population_count(keep)[0]
            plsc.store_compressed(cand_v.at[pl.ds(noff, _L)], vals, mask=keep)
            return noff + cnt

        nv = (off + _L - 1) // _L
        new_off = lax.fori_loop(0, nv, rbody, jnp.int32(0))
        new_m_v = jnp.full((_L,), _K - new_off, jnp.int32)
        return new_off, _unmono_f32(t_v), t_v, new_m_v

    def cheap_compact(off, thrf_v, thrm_v, m_v):
        q = (off // _L) // 4

        def cmax(c0):
            def b(j, acc):
                return jnp.maximum(acc, cand_v[pl.ds((c0 + j) * _L, _L)])

            return lax.fori_loop(1, q, b, cand_v[pl.ds(c0 * _L, _L)])

        mm = jnp.minimum(jnp.minimum(cmax(0), cmax(q)),
                         jnp.minimum(cmax(2 * q), cmax(3 * q)))
        tf_v = jnp.full((_L,), jnp.min(mm), jnp.float32)
        cand_v[pl.ds(off, _L)] = neginf_v
        nv = (off + _L - 1) // _L

        def rbody(j, noff):
            vals = cand_v[pl.ds(j * _L, _L)]
            keep = vals >= tf_v
            cnt = plsc.all_reduce_population_count(keep)[0]
            plsc.store_compressed(cand_v.at[pl.ds(noff, _L)], vals, mask=keep)
            return noff + cnt

        new_off = lax.fori_loop(0, nv, rbody, jnp.int32(0))
        return lax.cond(new_off >= _COMPACT_AT, exact_compact,
                        lambda o, tf, tm, mv: (o, tf, tm, mv),
                        new_off, tf_v, _mono_u32(tf_v), zero_i_v)

    def maybe_compact(off, thrf_v, thrm_v, m_v):
        return lax.cond(off >= _COMPACT_AT, cheap_compact,
                        lambda o, tf, tm, mv: (o, tf, tm, mv),
                        off, thrf_v, thrm_v, m_v)

    def filter_group(src, base, off, thrf_v):
        for u in range(_G):
            v = src[pl.ds((base + u) * _L, _L)]
            pmask = v > thrf_v
            cnt = plsc.all_reduce_population_count(pmask)[0]
            plsc.store_compressed(cand_v.at[pl.ds(off, _L)], v, mask=pmask)
            off = off + cnt
        return off

    def finalize(off, thrm_v, m_v):
        return _unmono_f32(bisect_v(off, thrm_v, m_v))

    for rr in range(_RPW):
        row = wid * _RPW + rr
        pltpu.sync_copy(x_hbm.at[row], row_v)

        @plsc.parallel_loop(0, _NG, step=1, unroll=2)
        def max_loop(g):
            acc = row_v[pl.ds(g * _G * _L, _L)]
            for u in range(1, _G):
                acc = jnp.maximum(acc, row_v[pl.ds((g * _G + u) * _L, _L)])
            max_v[pl.ds(g * _L, _L)] = acc

        def b_group(g, carry):
            off, thrf_v, thrm_v, m_v = carry
            off = filter_group(max_v, g * _G, off, thrf_v)
            return maybe_compact(off, thrf_v, thrm_v, m_v)

        off, thrf_v, thrm_v, m_v = lax.fori_loop(
            0, _NG // _G, b_group,
            (jnp.int32(0), neginf_v, zero_u_v, zero_i_v))
        s_mono_v = bisect_v(off, thrm_v, m_v)
        s_f_v = _unmono_f32(s_mono_v)

        def c_group(g, carry):
            off, thrf_v, thrm_v, m_v = carry
            hit = plsc.all_reduce_population_count(
                max_v[pl.ds(g * _L, _L)] > thrf_v)[0] > 0

            def slow(o, tf, tm, mv):
                o = filter_group(row_v, g * _G, o, tf)
                return maybe_compact(o, tf, tm, mv)

            return lax.cond(hit, slow, lambda o, tf, tm, mv: (o, tf, tm, mv),
                            off, thrf_v, thrm_v, m_v)

        off, thrf_v, thrm_v, m_v = lax.fori_loop(
            0, _NG, c_group, (jnp.int32(0), s_f_v, s_mono_v, k_v))

        tf = finalize(off, thrm_v, m_v)

        @plsc.parallel_loop(0, _NV, step=1, unroll=8)
        def mask_loop(j):
            v = row_v[pl.ds(j * _L, _L)]
            row_v[pl.ds(j * _L, _L)] = jnp.where(v >= tf, v, _NEG_INF)

        pltpu.sync_copy(row_v, out_hbm.at[row])


@jax.jit
def kernel(x):
    mesh = plsc.VectorSubcoreMesh(
        core_axis_name="c", subcore_axis_name="s",
        num_cores=_NC, num_subcores=_NS)
    run = pl.kernel(
        _body,
        out_type=jax.ShapeDtypeStruct((_R, _N), jnp.float32),
        mesh=mesh,
        scratch_types=[
            pltpu.VMEM((_N,), jnp.float32),
            pltpu.VMEM((_MV,), jnp.float32),
            pltpu.VMEM((_CAP + _L,), jnp.float32),
            pltpu.VMEM((_CAP + _L,), jnp.uint32),
        ],
        compiler_params=pltpu.CompilerParams(needs_layout_passes=False),
    )
    return run(x)


# speedup vs baseline: 1.5660x; 1.4229x over previous
import functools

import jax
import jax.numpy as jnp
import numpy as np
from jax import lax
from jax.experimental import pallas as pl
from jax.experimental.pallas import tpu as pltpu
from jax.experimental.pallas import tpu_sc as plsc

_R = 128
_N = 32768
_K = 64
_L = 16
_NV = _N // _L
_NC = 2
_NS = 16
_NW = _NC * _NS
_RPW = _R // _NW
_G = 8
_NG = _NV // _G
_MV = _NG * _L
_CAP = 512
_COMPACT_AT = _CAP - _G * _L

_NEG_INF = np.float32("-inf")


def _mono_u32(v):
    i = plsc.bitcast(v, jnp.int32)
    flip = lax.shift_right_arithmetic(i, 31) & jnp.int32(0x7FFFFFFF)
    mi = i ^ flip
    return plsc.bitcast(mi, jnp.uint32) ^ jnp.uint32(0x80000000)


def _unmono_f32(mu_vec):
    mi = plsc.bitcast(mu_vec ^ jnp.uint32(0x80000000), jnp.int32)
    flip = lax.shift_right_arithmetic(mi, 31) & jnp.int32(0x7FFFFFFF)
    return plsc.bitcast(mi ^ flip, jnp.float32)


def _body(x_hbm, out_hbm, row_v, max_v, cand_v, mono_v, gidx_v):
    cid = lax.axis_index("c")
    sid = lax.axis_index("s")
    wid = sid * _NC + cid

    lane = jnp.arange(_L, dtype=jnp.int32)
    lane0 = lane == 0
    neginf_v = jnp.full((_L,), _NEG_INF, jnp.float32)
    zero_u_v = jnp.zeros((_L,), jnp.uint32)
    zero_i_v = jnp.zeros((_L,), jnp.int32)
    one_i_v = jnp.full((_L,), 1, jnp.int32)
    k_v = jnp.full((_L,), _K, jnp.int32)

    def monoize(off):
        nv = (off + _L - 1) // _L

        def mbody(j, _):
            mono_v[pl.ds(j * _L, _L)] = _mono_u32(cand_v[pl.ds(j * _L, _L)])
            return _

        lax.fori_loop(0, nv, mbody, jnp.int32(0))
        mono_v[pl.ds(off, _L)] = zero_u_v

    def bisect_v(off, thrm_v, m_v):
        monoize(off)
        nv = (off + _L - 1) // _L
        one_v = jnp.full((_L,), jnp.uint32(1))

        def bit_body(i, t_v):
            shift = jnp.full((_L,), 31 - i, jnp.int32).astype(jnp.uint32)
            cand_vv = t_v | lax.shift_left(one_v, shift)

            def cbody(j, acc):
                ge = mono_v[pl.ds(j * _L, _L)] >= cand_vv
                return acc + plsc.all_reduce_population_count(ge)

            cnt = lax.fori_loop(0, nv, cbody, zero_i_v)
            cnt = cnt + jnp.where(cand_vv <= thrm_v, m_v, zero_i_v)
            return jnp.where(cnt >= k_v, cand_vv, t_v)

        return lax.fori_loop(0, 32, bit_body, zero_u_v)

    def scatter_keep(noff_v, vals, keep):
        pc = plsc.cumsum(jnp.where(keep, one_i_v, zero_i_v))
        idx = noff_v + pc - one_i_v
        plsc.store_scatter(cand_v, [idx], vals, mask=keep)
        return noff_v + plsc.all_reduce_population_count(keep)

    def exact_compact(off, thrf_v, thrm_v, m_v):
        t_v = bisect_v(off, thrm_v, m_v)
        nv = (off + _L - 1) // _L

        def rbody(j, noff_v):
            keep = mono_v[pl.ds(j * _L, _L)] > t_v
            vals = cand_v[pl.ds(j * _L, _L)]
            return scatter_keep(noff_v, vals, keep)

        new_off = lax.fori_loop(0, nv, rbody, zero_i_v)[0]
        new_m_v = jnp.full((_L,), _K - new_off, jnp.int32)
        return new_off, _unmono_f32(t_v), t_v, new_m_v

    def cheap_compact(off, thrf_v, thrm_v, m_v):
        q = (off // _L) // 4

        def cmax(c0):
            def b(j, acc):
                return jnp.maximum(acc, cand_v[pl.ds((c0 + j) * _L, _L)])

            return lax.fori_loop(1, q, b, cand_v[pl.ds(c0 * _L, _L)])

        mm = jnp.minimum(jnp.minimum(cmax(0), cmax(q)),
                         jnp.minimum(cmax(2 * q), cmax(3 * q)))
        tf_v = jnp.full((_L,), jnp.min(mm), jnp.float32)
        cand_v[pl.ds(off, _L)] = neginf_v
        nv = (off + _L - 1) // _L

        def rbody(j, noff_v):
            vals = cand_v[pl.ds(j * _L, _L)]
            return scatter_keep(noff_v, vals, vals >= tf_v)

        new_off = lax.fori_loop(0, nv, rbody, zero_i_v)[0]
        return lax.cond(new_off >= _COMPACT_AT, exact_compact,
                        lambda o, tf, tm, mv: (o, tf, tm, mv),
                        new_off, tf_v, _mono_u32(tf_v), zero_i_v)

    def maybe_compact(off_v, thrf_v, thrm_v, m_v):
        def do(o, tf, tm, mv):
            no, tf2, tm2, mv2 = cheap_compact(o, tf, tm, mv)
            return jnp.full((_L,), no, jnp.int32), tf2, tm2, mv2

        def skip(o, tf, tm, mv):
            return jnp.full((_L,), o, jnp.int32), tf, tm, mv

        off0 = off_v[0]
        return lax.cond(off0 >= _COMPACT_AT, do, skip,
                        off0, thrf_v, thrm_v, m_v)

    def filter_group(src, base, off_v, thrf_v):
        for u in range(_G):
            v = src[pl.ds((base + u) * _L, _L)]
            off_v = scatter_keep(off_v, v, v > thrf_v)
        return off_v

    for rr in range(_RPW):
        row = wid * _RPW + rr
        pltpu.sync_copy(x_hbm.at[row], row_v)

        @plsc.parallel_loop(0, _NG, step=1, unroll=2)
        def max_loop(g):
            acc = row_v[pl.ds(g * _G * _L, _L)]
            for u in range(1, _G):
                acc = jnp.maximum(acc, row_v[pl.ds((g * _G + u) * _L, _L)])
            max_v[pl.ds(g * _L, _L)] = acc

        def b_group(g, carry):
            off_v, thrf_v, thrm_v, m_v = carry
            off_v = filter_group(max_v, g * _G, off_v, thrf_v)
            return maybe_compact(off_v, thrf_v, thrm_v, m_v)

        off_v, thrf_v, thrm_v, m_v = lax.fori_loop(
            0, _NG // _G, b_group,
            (zero_i_v, neginf_v, zero_u_v, zero_i_v))
        s_mono_v = bisect_v(off_v[0], thrm_v, m_v)
        s_f_v = _unmono_f32(s_mono_v)

        def c1_body(g, goff_v):
            hit = max_v[pl.ds(g * _L, _L)] > s_f_v
            has = plsc.all_reduce_population_count(hit) > zero_i_v
            plsc.store_scatter(gidx_v, [goff_v],
                               jnp.full((_L,), g, jnp.int32),
                               mask=jnp.logical_and(has, lane0))
            return goff_v + jnp.where(has, one_i_v, zero_i_v)

        nhit = lax.fori_loop(0, _NG, c1_body, zero_i_v)[0]

        def c2_body(i, carry):
            off_v, thrf_v, thrm_v, m_v = carry
            g = gidx_v[pl.ds(i, _L)][0]
            off_v = filter_group(row_v, g * _G, off_v, thrf_v)
            return maybe_compact(off_v, thrf_v, thrm_v, m_v)

        off_v, thrf_v, thrm_v, m_v = lax.fori_loop(
            0, nhit, c2_body, (zero_i_v, s_f_v, s_mono_v, k_v))

        tf = _unmono_f32(bisect_v(off_v[0], thrm_v, m_v))

        @plsc.parallel_loop(0, _NV, step=1, unroll=8)
        def mask_loop(j):
            v = row_v[pl.ds(j * _L, _L)]
            row_v[pl.ds(j * _L, _L)] = jnp.where(v >= tf, v, _NEG_INF)

        pltpu.sync_copy(row_v, out_hbm.at[row])


@jax.jit
def kernel(x):
    mesh = plsc.VectorSubcoreMesh(
        core_axis_name="c", subcore_axis_name="s",
        num_cores=_NC, num_subcores=_NS)
    run = pl.kernel(
        _body,
        out_type=jax.ShapeDtypeStruct((_R, _N), jnp.float32),
        mesh=mesh,
        scratch_types=[
            pltpu.VMEM((_N,), jnp.float32),
            pltpu.VMEM((_MV,), jnp.float32),
            pltpu.VMEM((_CAP + _L,), jnp.float32),
            pltpu.VMEM((_CAP + _L,), jnp.uint32),
            pltpu.VMEM((_NG + _L,), jnp.int32),
        ],
        compiler_params=pltpu.CompilerParams(needs_layout_passes=False),
    )
    return run(x)


# speedup vs baseline: 1.7775x; 1.1350x over previous
import functools

import jax
import jax.numpy as jnp
import numpy as np
from jax import lax
from jax.experimental import pallas as pl
from jax.experimental.pallas import tpu as pltpu
from jax.experimental.pallas import tpu_sc as plsc

_R = 128
_N = 32768
_K = 64
_L = 16
_NV = _N // _L
_NC = 2
_NS = 16
_NW = _NC * _NS
_RPW = _R // _NW
_G = 8
_NG = _NV // _G
_MV = _NG * _L
_CAP = 1024
_COMPACT_AT = _CAP - _G * _L
_SHRINK_AT = 192

_NEG_INF = np.float32("-inf")


def _mono_u32(v):
    i = plsc.bitcast(v, jnp.int32)
    flip = lax.shift_right_arithmetic(i, 31) & jnp.int32(0x7FFFFFFF)
    mi = i ^ flip
    return plsc.bitcast(mi, jnp.uint32) ^ jnp.uint32(0x80000000)


def _unmono_f32(mu_vec):
    mi = plsc.bitcast(mu_vec ^ jnp.uint32(0x80000000), jnp.int32)
    flip = lax.shift_right_arithmetic(mi, 31) & jnp.int32(0x7FFFFFFF)
    return plsc.bitcast(mi ^ flip, jnp.float32)


def _body(x_hbm, out_hbm, row_v, max_v, cand_v, mono_v, gidx_v):
    cid = lax.axis_index("c")
    sid = lax.axis_index("s")
    wid = sid * _NC + cid

    lane = jnp.arange(_L, dtype=jnp.int32)
    lane0 = lane == 0
    neginf_v = jnp.full((_L,), _NEG_INF, jnp.float32)
    zero_u_v = jnp.zeros((_L,), jnp.uint32)
    zero_i_v = jnp.zeros((_L,), jnp.int32)
    one_i_v = jnp.full((_L,), 1, jnp.int32)
    k_v = jnp.full((_L,), _K, jnp.int32)

    def monoize(off):
        nv = (off + _L - 1) // _L

        @plsc.parallel_loop(0, nv, step=1, unroll=2)
        def mbody(j):
            mono_v[pl.ds(j * _L, _L)] = _mono_u32(cand_v[pl.ds(j * _L, _L)])

        mono_v[pl.ds(off, _L)] = zero_u_v

    def bisect_v(off, thrm_v, m_v):
        monoize(off)
        nv = (off + _L - 1) // _L
        one_v = jnp.full((_L,), jnp.uint32(1))

        def bit_body(i, t_v):
            shift = jnp.full((_L,), 31 - i, jnp.int32).astype(jnp.uint32)
            cand_vv = t_v | lax.shift_left(one_v, shift)

            @plsc.parallel_loop(0, nv, step=1, unroll=4, carry=zero_i_v)
            def cnt(j, acc):
                ge = mono_v[pl.ds(j * _L, _L)] >= cand_vv
                return acc + plsc.all_reduce_population_count(ge)

            cnt = cnt + jnp.where(cand_vv <= thrm_v, m_v, zero_i_v)
            return jnp.where(cnt >= k_v, cand_vv, t_v)

        return lax.fori_loop(0, 32, bit_body, zero_u_v)

    def scatter_keep(noff_v, vals, keep):
        pc = plsc.cumsum(jnp.where(keep, one_i_v, zero_i_v))
        idx = noff_v + pc - one_i_v
        plsc.store_scatter(cand_v, [idx], vals, mask=keep)
        return noff_v + plsc.all_reduce_population_count(keep)

    def exact_compact(off, thrf_v, thrm_v, m_v):
        t_v = bisect_v(off, thrm_v, m_v)
        nv = (off + _L - 1) // _L

        def rbody(j, noff_v):
            keep = mono_v[pl.ds(j * _L, _L)] > t_v
            vals = cand_v[pl.ds(j * _L, _L)]
            return scatter_keep(noff_v, vals, keep)

        new_off = lax.fori_loop(0, nv, rbody, zero_i_v)[0]
        new_m_v = jnp.full((_L,), _K - new_off, jnp.int32)
        return new_off, _unmono_f32(t_v), t_v, new_m_v

    def cheap_compact(off, thrf_v, thrm_v, m_v):
        q = (off // _L) // 4

        def cmax(c0):
            def b(j, acc):
                return jnp.maximum(acc, cand_v[pl.ds((c0 + j) * _L, _L)])

            return lax.fori_loop(1, q, b, cand_v[pl.ds(c0 * _L, _L)])

        mm = jnp.minimum(jnp.minimum(cmax(0), cmax(q)),
                         jnp.minimum(cmax(2 * q), cmax(3 * q)))
        tf_v = jnp.full((_L,), jnp.min(mm), jnp.float32)
        cand_v[pl.ds(off, _L)] = neginf_v
        nv = (off + _L - 1) // _L

        def rbody(j, noff_v):
            vals = cand_v[pl.ds(j * _L, _L)]
            return scatter_keep(noff_v, vals, vals >= tf_v)

        new_off = lax.fori_loop(0, nv, rbody, zero_i_v)[0]
        return lax.cond(new_off >= _COMPACT_AT, exact_compact,
                        lambda o, tf, tm, mv: (o, tf, tm, mv),
                        new_off, tf_v, _mono_u32(tf_v), zero_i_v)

    def maybe_compact(off_v, thrf_v, thrm_v, m_v):
        def do(o, tf, tm, mv):
            no, tf2, tm2, mv2 = cheap_compact(o, tf, tm, mv)
            return jnp.full((_L,), no, jnp.int32), tf2, tm2, mv2

        def skip(o, tf, tm, mv):
            return jnp.full((_L,), o, jnp.int32), tf, tm, mv

        off0 = off_v[0]
        return lax.cond(off0 >= _COMPACT_AT, do, skip,
                        off0, thrf_v, thrm_v, m_v)

    def filter_group(src, base, off_v, thrf_v):
        for u in range(_G):
            v = src[pl.ds((base + u) * _L, _L)]
            off_v = scatter_keep(off_v, v, v > thrf_v)
        return off_v

    def select_kth(off0, thrf_v, thrm_v, m_v):
        off0, thrf_v, thrm_v, m_v = lax.cond(
            off0 >= _SHRINK_AT, cheap_compact,
            lambda o, tf, tm, mv: (o, tf, tm, mv),
            off0, thrf_v, thrm_v, m_v)
        return bisect_v(off0, thrm_v, m_v)

    for rr in range(_RPW):
        row = wid * _RPW + rr
        pltpu.sync_copy(x_hbm.at[row], row_v)

        @plsc.parallel_loop(0, _NG, step=1, unroll=2)
        def max_loop(g):
            acc = row_v[pl.ds(g * _G * _L, _L)]
            for u in range(1, _G):
                acc = jnp.maximum(acc, row_v[pl.ds((g * _G + u) * _L, _L)])
            max_v[pl.ds(g * _L, _L)] = acc

        def b_group(g, carry):
            off_v, thrf_v, thrm_v, m_v = carry
            off_v = filter_group(max_v, g * _G, off_v, thrf_v)
            return maybe_compact(off_v, thrf_v, thrm_v, m_v)

        off_v, thrf_v, thrm_v, m_v = lax.fori_loop(
            0, _NG // _G, b_group,
            (zero_i_v, neginf_v, zero_u_v, zero_i_v))
        s_mono_v = select_kth(off_v[0], thrf_v, thrm_v, m_v)
        s_f_v = _unmono_f32(s_mono_v)

        @plsc.parallel_loop(0, _NG, step=1, unroll=4, carry=zero_i_v)
        def goff_v(g, goff):
            hit = max_v[pl.ds(g * _L, _L)] > s_f_v
            has = plsc.all_reduce_population_count(hit) > zero_i_v
            plsc.store_scatter(gidx_v, [goff],
                               jnp.full((_L,), g, jnp.int32),
                               mask=jnp.logical_and(has, lane0))
            return goff + jnp.where(has, one_i_v, zero_i_v)

        nhit = goff_v[0]

        def c2_body(i, off_v):
            g = gidx_v[pl.ds(i, _L)][0]
            return filter_group(row_v, g * _G, off_v, s_f_v)

        off_v = lax.fori_loop(0, nhit, c2_body, zero_i_v)

        tf = _unmono_f32(select_kth(off_v[0], s_f_v, s_mono_v, k_v))

        @plsc.parallel_loop(0, _NV, step=1, unroll=8)
        def mask_loop(j):
            v = row_v[pl.ds(j * _L, _L)]
            row_v[pl.ds(j * _L, _L)] = jnp.where(v >= tf, v, _NEG_INF)

        pltpu.sync_copy(row_v, out_hbm.at[row])


@jax.jit
def kernel(x):
    mesh = plsc.VectorSubcoreMesh(
        core_axis_name="c", subcore_axis_name="s",
        num_cores=_NC, num_subcores=_NS)
    run = pl.kernel(
        _body,
        out_type=jax.ShapeDtypeStruct((_R, _N), jnp.float32),
        mesh=mesh,
        scratch_types=[
            pltpu.VMEM((_N,), jnp.float32),
            pltpu.VMEM((_MV,), jnp.float32),
            pltpu.VMEM((_CAP + _L,), jnp.float32),
            pltpu.VMEM((_CAP + _L,), jnp.uint32),
            pltpu.VMEM((_NG + _L,), jnp.int32),
        ],
        compiler_params=pltpu.CompilerParams(needs_layout_passes=False),
    )
    return run(x)
